# baseline (speedup 1.0000x reference)
import functools

import jax
import jax.numpy as jnp
from jax import lax
from jax.experimental import pallas as pl
from jax.experimental.pallas import tpu as pltpu
from jax.experimental.pallas import tpu_sc as plsc

NUM_EMB = 1000000
DIM = 32
QMAX = 255.0
MAGIC = float(1.5 * 2**23)

ROWS_PER_STEP = 8000
MINMAX_GRID = NUM_EMB // ROWS_PER_STEP


def _minmax_body(w_ref, scale_ref, zp_ref, mn_ref, mx_ref):
    i = pl.program_id(0)

    @pl.when(i == 0)
    def _init():
        mn_ref[0] = 0.0
        mx_ref[0] = 0.0

    w = w_ref[...]
    mn_ref[0] = jnp.minimum(mn_ref[0], jnp.min(w))
    mx_ref[0] = jnp.maximum(mx_ref[0], jnp.max(w))

    @pl.when(i == MINMAX_GRID - 1)
    def _finish():
        mn = mn_ref[0]
        mx = mx_ref[0]
        scale = jnp.maximum((mx - mn) / QMAX, 1e-8)
        zp = jnp.clip(jnp.round(-mn / scale), 0.0, QMAX)
        scale_ref[...] = jnp.full((1, 128), scale, jnp.float32)
        zp_ref[...] = jnp.full((1, 128), zp, jnp.float32)


_quant_params = pl.pallas_call(
    _minmax_body,
    grid=(MINMAX_GRID,),
    in_specs=[pl.BlockSpec((ROWS_PER_STEP, DIM), lambda i: (i, 0))],
    out_specs=[
        pl.BlockSpec((1, 128), lambda i: (0, 0)),
        pl.BlockSpec((1, 128), lambda i: (0, 0)),
    ],
    out_shape=[
        jax.ShapeDtypeStruct((1, 128), jnp.float32),
        jax.ShapeDtypeStruct((1, 128), jnp.float32),
    ],
    scratch_shapes=[
        pltpu.SMEM((1,), jnp.float32),
        pltpu.SMEM((1,), jnp.float32),
    ],
)

B_TOTAL = 4096 * 50
NUM_WORKERS = 32
B_PER_W = B_TOTAL // NUM_WORKERS
CHUNK = 1280
NCHUNK = B_PER_W // CHUNK
SUB = 128
NSUB = CHUNK // SUB

_sc_mesh = plsc.VectorSubcoreMesh(core_axis_name="c", subcore_axis_name="s")


@functools.partial(
    pl.kernel,
    mesh=_sc_mesh,
    out_type=jax.ShapeDtypeStruct((B_TOTAL, DIM), jnp.float32),
    scratch_types=[
        pltpu.VMEM((CHUNK,), jnp.int32),
        pltpu.VMEM((CHUNK, DIM), jnp.float32),
        pltpu.VMEM((CHUNK, 8), jnp.int32),
        pltpu.VMEM((16,), jnp.float32),
        pltpu.VMEM((16,), jnp.float32),
        pltpu.SemaphoreType.DMA,
    ],
    compiler_params=pltpu.CompilerParams(
        needs_layout_passes=False, use_tc_tiling_on_sc=False),
)
def _sc_lookup(idx_hbm, w_hbm, m_hbm, scale_hbm, zp_hbm, out_hbm,
               idx_v, w_v, m_v, scale_v, zp_v, sem):
    wid = lax.axis_index("s") * 2 + lax.axis_index("c")
    base = wid * B_PER_W

    pltpu.sync_copy(scale_hbm.at[pl.ds(0, 16)], scale_v)
    pltpu.sync_copy(zp_hbm.at[pl.ds(0, 16)], zp_v)
    s = scale_v[...]
    zp = zp_v[...]
    inv = 1.0 / s
    lo = -s * zp
    hi = s * (QMAX - zp)
    magic = jnp.full((16,), MAGIC, jnp.float32)

    iota = lax.iota(jnp.int32, 16)
    widx0 = iota >> 2
    shamt = (iota & 3) * 8

    def do_chunk(c, carry):
        off = base + c * CHUNK
        pltpu.sync_copy(idx_hbm.at[pl.ds(off, CHUNK)], idx_v)
        cps = []
        for sub in range(NSUB):
            isl = idx_v.at[pl.ds(sub * SUB, SUB)]
            cps.append(pltpu.async_copy(
                w_hbm.at[isl], w_v.at[pl.ds(sub * SUB, SUB)], sem))
            cps.append(pltpu.async_copy(
                m_hbm.at[isl], m_v.at[pl.ds(sub * SUB, SUB)], sem))
        for cp in cps:
            cp.wait()

        def do_row(r, carry2):
            rfull = jnp.full((16,), r, jnp.int32)
            for j in range(2):
                w = w_v[r, pl.ds(j * 16, 16)]
                words = plsc.load_gather(m_v, [rfull, j * 4 + widx0])
                mbyte = (words >> shamt) & 255
                t = w * inv + zp
                rr = (t + magic) - magic
                q = jnp.clip(rr, 0.0, QMAX)
                wq = (q - zp) * s
                noise = jnp.where(mbyte == 0, wq - w, 0.0)
                w_v[r, pl.ds(j * 16, 16)] = jnp.clip(w, lo, hi) + noise
            return carry2

        lax.fori_loop(0, CHUNK, do_row, 0)
        pltpu.sync_copy(w_v, out_hbm.at[pl.ds(off, CHUNK)])
        return carry

    lax.fori_loop(0, NCHUNK, do_chunk, 0)


def kernel(input, weight, mask):
    scale_r, zp_r = _quant_params(weight)
    idx = input.reshape(-1)
    mask_i32 = lax.bitcast_convert_type(
        mask.astype(jnp.uint8).reshape(NUM_EMB, 8, 4), jnp.int32)
    out = _sc_lookup(idx, weight, mask_i32,
                     scale_r.reshape(-1), zp_r.reshape(-1))
    return out.reshape(input.shape + (DIM,))
